# baseline (speedup 1.0000x reference)
import functools

import jax
import jax.numpy as jnp
import numpy as np
from jax import lax
from jax.experimental import pallas as pl
from jax.experimental.pallas import tpu as pltpu
from jax.experimental.pallas import tpu_sc as plsc

B = 2048
D = 1024
H = 4096
O = 1024
E = 8
K = 2
T = 1.0

M = 768
NB = 13
P_MAX = NB * M
HB = 2048
NH = H // HB

NW = 32
_EPS = float(np.finfo(np.float64).eps)

_G_ROWS = P_MAX // NW
_G_CH = 24
_G_NCH = _G_ROWS // _G_CH

_C_TOK = B // NW
_C_CH = 16
_C_NCH = _C_TOK // _C_CH


def _sc_gather_body(x_hbm, idx_hbm, out_hbm, idx_v, rows_a, rows_b, sa, sb):
    wid = lax.axis_index("s") * 2 + lax.axis_index("c")
    base = wid * _G_ROWS
    pltpu.sync_copy(idx_hbm.at[pl.ds(base, _G_ROWS)], idx_v)
    bufs = (rows_a, rows_b)
    sems = (sa, sb)
    cps = [None, None]

    def fire(c):
        return pltpu.async_copy(
            x_hbm.at[idx_v.at[pl.ds(c * _G_CH, _G_CH)]],
            bufs[c % 2], sems[c % 2])

    cps[0] = fire(0)
    for c in range(_G_NCH):
        if c + 1 < _G_NCH:
            cps[(c + 1) % 2] = fire(c + 1)
        cps[c % 2].wait()
        pltpu.sync_copy(bufs[c % 2],
                        out_hbm.at[pl.ds(base + c * _G_CH, _G_CH)])


def _sc_combine_body(rows_hbm, pos0_hbm, pos1_hbm, y_hbm, p0_v, p1_v,
                     ra0, ra1, rb0, rb1, sa0, sa1, sb0, sb1, sw0, sw1):
    wid = lax.axis_index("s") * 2 + lax.axis_index("c")
    base = wid * _C_TOK
    pltpu.sync_copy(pos0_hbm.at[pl.ds(base, _C_TOK)], p0_v)
    pltpu.sync_copy(pos1_hbm.at[pl.ds(base, _C_TOK)], p1_v)
    pairs = ((ra0, ra1, sa0, sa1, sw0), (rb0, rb1, sb0, sb1, sw1))
    cps = [None, None]
    wcps = [None, None]

    def fire(c):
        r0b, r1b, s0b, s1b, _ = pairs[c % 2]
        return (
            pltpu.async_copy(
                rows_hbm.at[p0_v.at[pl.ds(c * _C_CH, _C_CH)]], r0b, s0b),
            pltpu.async_copy(
                rows_hbm.at[p1_v.at[pl.ds(c * _C_CH, _C_CH)]], r1b, s1b),
        )

    cps[0] = fire(0)
    for c in range(_C_NCH):
        if c + 1 < _C_NCH:
            if wcps[(c + 1) % 2] is not None:
                wcps[(c + 1) % 2].wait()
                wcps[(c + 1) % 2] = None
            cps[(c + 1) % 2] = fire(c + 1)
        cp0, cp1 = cps[c % 2]
        cp0.wait()
        cp1.wait()
        r0b, r1b = pairs[c % 2][0], pairs[c % 2][1]

        def row_body(r, _):
            for g in range(O // 16):
                v = r0b[r, pl.ds(g * 16, 16)] + r1b[r, pl.ds(g * 16, 16)]
                v = jnp.where(v == 0.0, _EPS, v)
                r0b[r, pl.ds(g * 16, 16)] = v
            return 0

        lax.fori_loop(0, _C_CH, row_body, 0)
        wcps[c % 2] = pltpu.async_copy(
            r0b, y_hbm.at[pl.ds(base + c * _C_CH, _C_CH)], pairs[c % 2][4])
    for w in wcps:
        if w is not None:
            w.wait()


@functools.lru_cache(maxsize=None)
def _sc_kernels():
    mesh = plsc.VectorSubcoreMesh(core_axis_name="c", subcore_axis_name="s")
    gather = pl.kernel(
        _sc_gather_body,
        mesh=mesh,
        out_type=jax.ShapeDtypeStruct((P_MAX, D), jnp.float32),
        scratch_types=[
            pltpu.VMEM((_G_ROWS,), jnp.int32),
            pltpu.VMEM((_G_CH, D), jnp.float32),
            pltpu.VMEM((_G_CH, D), jnp.float32),
            pltpu.SemaphoreType.DMA,
            pltpu.SemaphoreType.DMA,
        ],
    )
    combine = pl.kernel(
        _sc_combine_body,
        mesh=mesh,
        out_type=jax.ShapeDtypeStruct((B, O), jnp.float32),
        scratch_types=[
            pltpu.VMEM((_C_TOK,), jnp.int32),
            pltpu.VMEM((_C_TOK,), jnp.int32),
            pltpu.VMEM((_C_CH, O), jnp.float32),
            pltpu.VMEM((_C_CH, O), jnp.float32),
            pltpu.VMEM((_C_CH, O), jnp.float32),
            pltpu.VMEM((_C_CH, O), jnp.float32),
            pltpu.SemaphoreType.DMA,
            pltpu.SemaphoreType.DMA,
            pltpu.SemaphoreType.DMA,
            pltpu.SemaphoreType.DMA,
            pltpu.SemaphoreType.DMA,
            pltpu.SemaphoreType.DMA,
        ],
    )
    return gather, combine


def _sc_gather_rows(x, row_ids):
    return _sc_kernels()[0](x, row_ids)


def _sc_combine(rows, pos0, pos1):
    return _sc_kernels()[1](rows, pos0, pos1)


def _mlp_body(eb_ref, vb_ref, fi_ref, jt_ref, xs_ref, w1_ref, b1_ref, w2_ref,
              b2_ref, g_ref, out_ref, xb_ref):
    i = pl.program_id(0)
    j = pl.program_id(1)

    @pl.when(vb_ref[i] == 1)
    def _():
        @pl.when(j == 0)
        def _():
            xb_ref[...] = xs_ref[...].astype(jnp.bfloat16)

        xb = xb_ref[...]
        w1 = w1_ref[0].astype(jnp.bfloat16)
        h = jnp.dot(xb, w1, preferred_element_type=jnp.float32) + b1_ref[0]
        h = jnp.maximum(h, 0.0).astype(jnp.bfloat16)
        w2 = w2_ref[0].astype(jnp.bfloat16)
        part = jnp.dot(h, w2, preferred_element_type=jnp.float32)

        @pl.when(j == 0)
        def _():
            out_ref[...] = part

        @pl.when(j > 0)
        def _():
            out_ref[...] += part

        @pl.when(j == NH - 1)
        def _():
            out_ref[...] = (out_ref[...] + b2_ref[0]) * g_ref[...]


_MLP_GRID = pltpu.PrefetchScalarGridSpec(
    num_scalar_prefetch=4,
    grid=(NB, NH),
    in_specs=[
        pl.BlockSpec((M, D), lambda i, j, eb, vb, fi, jt: (fi[i], 0)),
        pl.BlockSpec((1, D, HB),
                     lambda i, j, eb, vb, fi, jt: (eb[i], 0, jt[i, j])),
        pl.BlockSpec((1, 1, HB),
                     lambda i, j, eb, vb, fi, jt: (eb[i], 0, jt[i, j])),
        pl.BlockSpec((1, HB, O),
                     lambda i, j, eb, vb, fi, jt: (eb[i], jt[i, j], 0)),
        pl.BlockSpec((1, 1, O), lambda i, j, eb, vb, fi, jt: (eb[i], 0, 0)),
        pl.BlockSpec((M, 1), lambda i, j, eb, vb, fi, jt: (fi[i], 0)),
    ],
    out_specs=pl.BlockSpec((M, O), lambda i, j, eb, vb, fi, jt: (fi[i], 0)),
    scratch_shapes=[pltpu.VMEM((M, D), jnp.bfloat16)],
)

_mlp_call = pl.pallas_call(
    _mlp_body,
    grid_spec=_MLP_GRID,
    out_shape=jax.ShapeDtypeStruct((P_MAX, O), jnp.float32),
)


def _cv_sq(v):
    eps = 1e-10
    v = v.astype(jnp.float32)
    return jnp.var(v, ddof=1) / (jnp.mean(v) ** 2 + eps)


def kernel(x, w_gate, W1, b1, W2, b2):
    logits = x @ w_gate
    top_logits, top_idx = jax.lax.top_k(logits, K)
    top_gates = jax.nn.softmax(top_logits / T, axis=-1)

    flat_e = top_idx.reshape(-1).astype(jnp.int32)
    onehot = (flat_e[:, None] ==
              jnp.arange(E, dtype=jnp.int32)[None, :]).astype(jnp.int32)
    counts = onehot.sum(0)

    importance = (top_gates.reshape(-1)[:, None] *
                  onehot.astype(jnp.float32)).sum(0)
    load = counts
    loss = (_cv_sq(importance) + _cv_sq(load)) * 1e-2
    within = jnp.cumsum(onehot, axis=0) - onehot
    rank = (within * onehot).sum(1)
    padded = ((counts + M - 1) // M) * M
    poffsets = jnp.cumsum(padded) - padded
    dest = (poffsets[flat_e] + rank).astype(jnp.int32)
    tok = jnp.arange(B * K, dtype=jnp.int32) // K
    row_ids = (jnp.arange(P_MAX, dtype=jnp.int32) % B).at[dest].set(tok)
    gs = jnp.zeros((P_MAX,), jnp.float32).at[dest].set(top_gates.reshape(-1))
    gs2 = gs[:, None]
    pos0 = dest[0::K]
    pos1 = dest[1::K]
    pend = jnp.cumsum(padded)
    starts = jnp.arange(NB, dtype=jnp.int32) * M
    block_expert = jnp.searchsorted(pend, starts, side="right").astype(
        jnp.int32)
    is_valid = block_expert < E
    block_valid = is_valid.astype(jnp.int32)
    lastv = (pend[-1] // M - 1).astype(jnp.int32)
    block_expert = jnp.minimum(block_expert, E - 1)
    block_expert = jnp.where(is_valid, block_expert, block_expert[lastv])
    frozen_i = jnp.where(is_valid, jnp.arange(NB, dtype=jnp.int32), lastv)
    jseq = jnp.arange(NH, dtype=jnp.int32)[None, :]
    ii = jnp.arange(NB, dtype=jnp.int32)[:, None]
    zig = jnp.where(ii % 2 == 0, jseq, NH - 1 - jseq)
    frozen_j = jnp.where(lastv % 2 == 0, NH - 1, 0)
    jtab = jnp.where(is_valid[:, None], zig, frozen_j).astype(jnp.int32)

    xs = _sc_gather_rows(x, row_ids)
    b1r = b1.reshape(E, 1, H)
    b2r = b2.reshape(E, 1, O)
    out_sorted = _mlp_call(block_expert, block_valid, frozen_i, jtab, xs, W1,
                           b1r, W2, b2r, gs2)
    y = _sc_combine(out_sorted, pos0, pos1)
    return y, loss


# speedup vs baseline: 1.0673x; 1.0673x over previous
import functools

import jax
import jax.numpy as jnp
import numpy as np
from jax import lax
from jax.experimental import pallas as pl
from jax.experimental.pallas import tpu as pltpu
from jax.experimental.pallas import tpu_sc as plsc

B = 2048
D = 1024
H = 4096
O = 1024
E = 8
K = 2
T = 1.0

M = 640
NB = 14
P_MAX = NB * M
HB = 2048
NH = H // HB

NW = 32
_EPS = float(np.finfo(np.float64).eps)

_G_ROWS = P_MAX // NW
_G_CH = 40
_G_NCH = _G_ROWS // _G_CH

_C_TOK = B // NW
_C_CH = 16
_C_NCH = _C_TOK // _C_CH


def _sc_gather_body(x_hbm, idx_hbm, out_hbm, idx_v, rows_a, rows_b, sa, sb):
    wid = lax.axis_index("s") * 2 + lax.axis_index("c")
    base = wid * _G_ROWS
    pltpu.sync_copy(idx_hbm.at[pl.ds(base, _G_ROWS)], idx_v)
    bufs = (rows_a, rows_b)
    sems = (sa, sb)
    cps = [None, None]

    def fire(c):
        return pltpu.async_copy(
            x_hbm.at[idx_v.at[pl.ds(c * _G_CH, _G_CH)]],
            bufs[c % 2], sems[c % 2])

    cps[0] = fire(0)
    for c in range(_G_NCH):
        if c + 1 < _G_NCH:
            cps[(c + 1) % 2] = fire(c + 1)
        cps[c % 2].wait()
        pltpu.sync_copy(bufs[c % 2],
                        out_hbm.at[pl.ds(base + c * _G_CH, _G_CH)])


def _sc_combine_body(rows_hbm, pos0_hbm, pos1_hbm, y_hbm, p0_v, p1_v,
                     ra0, ra1, rb0, rb1, sa0, sa1, sb0, sb1):
    wid = lax.axis_index("s") * 2 + lax.axis_index("c")
    base = wid * _C_TOK
    pltpu.sync_copy(pos0_hbm.at[pl.ds(base, _C_TOK)], p0_v)
    pltpu.sync_copy(pos1_hbm.at[pl.ds(base, _C_TOK)], p1_v)
    pairs = ((ra0, ra1, sa0, sa1), (rb0, rb1, sb0, sb1))
    cps = [None, None]

    def fire(c):
        r0b, r1b, s0b, s1b = pairs[c % 2]
        return (
            pltpu.async_copy(
                rows_hbm.at[p0_v.at[pl.ds(c * _C_CH, _C_CH)]], r0b, s0b),
            pltpu.async_copy(
                rows_hbm.at[p1_v.at[pl.ds(c * _C_CH, _C_CH)]], r1b, s1b),
        )

    cps[0] = fire(0)
    for c in range(_C_NCH):
        if c + 1 < _C_NCH:
            cps[(c + 1) % 2] = fire(c + 1)
        cp0, cp1 = cps[c % 2]
        cp0.wait()
        cp1.wait()
        r0b, r1b = pairs[c % 2][0], pairs[c % 2][1]

        def row_body(r, _):
            for g in range(O // 16):
                v = r0b[r, pl.ds(g * 16, 16)] + r1b[r, pl.ds(g * 16, 16)]
                v = jnp.where(v == 0.0, _EPS, v)
                r0b[r, pl.ds(g * 16, 16)] = v
            return 0

        lax.fori_loop(0, _C_CH, row_body, 0)
        pltpu.sync_copy(r0b, y_hbm.at[pl.ds(base + c * _C_CH, _C_CH)])


@functools.lru_cache(maxsize=None)
def _sc_kernels():
    mesh = plsc.VectorSubcoreMesh(core_axis_name="c", subcore_axis_name="s")
    gather = pl.kernel(
        _sc_gather_body,
        mesh=mesh,
        out_type=jax.ShapeDtypeStruct((P_MAX, D), jnp.float32),
        scratch_types=[
            pltpu.VMEM((_G_ROWS,), jnp.int32),
            pltpu.VMEM((_G_CH, D), jnp.float32),
            pltpu.VMEM((_G_CH, D), jnp.float32),
            pltpu.SemaphoreType.DMA,
            pltpu.SemaphoreType.DMA,
        ],
    )
    combine = pl.kernel(
        _sc_combine_body,
        mesh=mesh,
        out_type=jax.ShapeDtypeStruct((B, O), jnp.float32),
        scratch_types=[
            pltpu.VMEM((_C_TOK,), jnp.int32),
            pltpu.VMEM((_C_TOK,), jnp.int32),
            pltpu.VMEM((_C_CH, O), jnp.float32),
            pltpu.VMEM((_C_CH, O), jnp.float32),
            pltpu.VMEM((_C_CH, O), jnp.float32),
            pltpu.VMEM((_C_CH, O), jnp.float32),
            pltpu.SemaphoreType.DMA,
            pltpu.SemaphoreType.DMA,
            pltpu.SemaphoreType.DMA,
            pltpu.SemaphoreType.DMA,
        ],
    )
    return gather, combine


def _sc_gather_rows(x, row_ids):
    return _sc_kernels()[0](x, row_ids)


def _sc_combine(rows, pos0, pos1):
    return _sc_kernels()[1](rows, pos0, pos1)


def _mlp_body(eb_ref, vb_ref, fi_ref, jt_ref, xs_ref, w1_ref, b1_ref, w2_ref,
              b2_ref, g_ref, out_ref):
    i = pl.program_id(0)
    j = pl.program_id(1)

    @pl.when(vb_ref[i] == 1)
    def _():
        xb = xs_ref[...].astype(jnp.bfloat16)
        w1 = w1_ref[0].astype(jnp.bfloat16)
        h = jnp.dot(xb, w1, preferred_element_type=jnp.float32) + b1_ref[0]
        h = jnp.maximum(h, 0.0).astype(jnp.bfloat16)
        w2 = w2_ref[0].astype(jnp.bfloat16)
        part = jnp.dot(h, w2, preferred_element_type=jnp.float32)

        @pl.when(j == 0)
        def _():
            out_ref[...] = part

        @pl.when(j > 0)
        def _():
            out_ref[...] += part

        @pl.when(j == NH - 1)
        def _():
            out_ref[...] = (out_ref[...] + b2_ref[0]) * g_ref[...]


_MLP_GRID = pltpu.PrefetchScalarGridSpec(
    num_scalar_prefetch=4,
    grid=(NB, NH),
    in_specs=[
        pl.BlockSpec((M, D), lambda i, j, eb, vb, fi, jt: (fi[i], 0)),
        pl.BlockSpec((1, D, HB),
                     lambda i, j, eb, vb, fi, jt: (eb[i], 0, jt[i, j])),
        pl.BlockSpec((1, 1, HB),
                     lambda i, j, eb, vb, fi, jt: (eb[i], 0, jt[i, j])),
        pl.BlockSpec((1, HB, O),
                     lambda i, j, eb, vb, fi, jt: (eb[i], jt[i, j], 0)),
        pl.BlockSpec((1, 1, O), lambda i, j, eb, vb, fi, jt: (eb[i], 0, 0)),
        pl.BlockSpec((M, 1), lambda i, j, eb, vb, fi, jt: (fi[i], 0)),
    ],
    out_specs=pl.BlockSpec((M, O), lambda i, j, eb, vb, fi, jt: (fi[i], 0)),
)

_mlp_call = pl.pallas_call(
    _mlp_body,
    grid_spec=_MLP_GRID,
    out_shape=jax.ShapeDtypeStruct((P_MAX, O), jnp.float32),
)


def _cv_sq(v):
    eps = 1e-10
    v = v.astype(jnp.float32)
    return jnp.var(v, ddof=1) / (jnp.mean(v) ** 2 + eps)


def kernel(x, w_gate, W1, b1, W2, b2):
    logits = x @ w_gate
    top_logits, top_idx = jax.lax.top_k(logits, K)
    top_gates = jax.nn.softmax(top_logits / T, axis=-1)

    flat_e = top_idx.reshape(-1).astype(jnp.int32)
    onehot = (flat_e[:, None] ==
              jnp.arange(E, dtype=jnp.int32)[None, :]).astype(jnp.int32)
    counts = onehot.sum(0)

    importance = (top_gates.reshape(-1)[:, None] *
                  onehot.astype(jnp.float32)).sum(0)
    load = counts
    loss = (_cv_sq(importance) + _cv_sq(load)) * 1e-2
    within = jnp.cumsum(onehot, axis=0) - onehot
    rank = (within * onehot).sum(1)
    padded = ((counts + M - 1) // M) * M
    poffsets = jnp.cumsum(padded) - padded
    dest = (poffsets[flat_e] + rank).astype(jnp.int32)
    tok = jnp.arange(B * K, dtype=jnp.int32) // K
    row_ids = (jnp.arange(P_MAX, dtype=jnp.int32) % B).at[dest].set(tok)
    gs = jnp.zeros((P_MAX,), jnp.float32).at[dest].set(top_gates.reshape(-1))
    gs2 = gs[:, None]
    pos0 = dest[0::K]
    pos1 = dest[1::K]
    pend = jnp.cumsum(padded)
    starts = jnp.arange(NB, dtype=jnp.int32) * M
    block_expert = jnp.searchsorted(pend, starts, side="right").astype(
        jnp.int32)
    is_valid = block_expert < E
    block_valid = is_valid.astype(jnp.int32)
    lastv = (pend[-1] // M - 1).astype(jnp.int32)
    block_expert = jnp.minimum(block_expert, E - 1)
    block_expert = jnp.where(is_valid, block_expert, block_expert[lastv])
    frozen_i = jnp.where(is_valid, jnp.arange(NB, dtype=jnp.int32), lastv)
    jseq = jnp.arange(NH, dtype=jnp.int32)[None, :]
    ii = jnp.arange(NB, dtype=jnp.int32)[:, None]
    zig = jnp.where(ii % 2 == 0, jseq, NH - 1 - jseq)
    frozen_j = jnp.where(lastv % 2 == 0, NH - 1, 0)
    jtab = jnp.where(is_valid[:, None], zig, frozen_j).astype(jnp.int32)

    xs = _sc_gather_rows(x, row_ids)
    b1r = b1.reshape(E, 1, H)
    b2r = b2.reshape(E, 1, O)
    out_sorted = _mlp_call(block_expert, block_valid, frozen_i, jtab, xs, W1,
                           b1r, W2, b2r, gs2)
    y = _sc_combine(out_sorted, pos0, pos1)
    return y, loss
